# Initial kernel scaffold; baseline (speedup 1.0000x reference)
import jax
import jax.numpy as jnp
from jax.experimental import pallas as pl


def kernel(hidden_states, Wq, Wk, Wv, Wo, v_emb0, v_emb1):
    raise NotImplementedError("write your pallas kernel here")


# speedup vs baseline: 3.4018x; 3.4018x over previous
import jax
import jax.numpy as jnp
from jax.experimental import pallas as pl
from jax.experimental.pallas import tpu as pltpu

_H = 8
_KVH = 2
_QKB = 8
_VB = 16
_TAU = 1.0
_BQ = 256


def _rosa_kernel(hs_ref, wq_ref, wk_ref, wv_ref, wo_ref, ve0_ref, ve1_ref,
                 out_ref, kb0_ref, kb1_ref, vb0_ref, vb1_ref):
    qi = pl.program_id(0)
    S = hs_ref.shape[0]

    @pl.when(qi == 0)
    def _project_kv():
        hs = hs_ref[...]
        kb = jax.nn.sigmoid(
            jnp.dot(hs, wk_ref[...], preferred_element_type=jnp.float32) / _TAU)
        vb = jax.nn.sigmoid(
            jnp.dot(hs, wv_ref[...], preferred_element_type=jnp.float32) / _TAU)
        pad = jnp.zeros((S, 16 - _QKB - 1), jnp.float32)
        for g, kref, vref in ((0, kb0_ref, vb0_ref), (1, kb1_ref, vb1_ref)):
            kbg = kb[:, g * _QKB:(g + 1) * _QKB]
            ksum = jnp.sum(kbg, axis=1, keepdims=True)
            kref[...] = jnp.concatenate([2.0 * kbg, -ksum, pad], axis=1)
            vref[...] = vb[:, g * _VB:(g + 1) * _VB]

    hq = hs_ref[pl.ds(qi * _BQ, _BQ), :]
    qb = jax.nn.sigmoid(
        jnp.dot(hq, wq_ref[...], preferred_element_type=jnp.float32) / _TAU)

    row = qi * _BQ + jax.lax.broadcasted_iota(jnp.int32, (_BQ, S), 0)
    col = jax.lax.broadcasted_iota(jnp.int32, (_BQ, S), 1)
    mask = col <= row

    ones = jnp.ones((_BQ, 1), jnp.float32)
    zpad = jnp.zeros((_BQ, 16 - _QKB - 1), jnp.float32)
    obits = []
    for h in range(_H):
        g = h // (_H // _KVH)
        kbp = (kb0_ref if g == 0 else kb1_ref)[...]
        vbg = (vb0_ref if g == 0 else vb1_ref)[...]
        qh = jnp.concatenate(
            [qb[:, h * _QKB:(h + 1) * _QKB], ones, zpad], axis=1)
        s = jax.lax.dot_general(qh, kbp, (((1,), (1,)), ((), ())),
                                preferred_element_type=jnp.float32)
        p = jnp.where(mask, jnp.exp(s - 16.0), 0.0)
        den = jnp.sum(p, axis=1, keepdims=True)
        o = jnp.dot(p, vbg, preferred_element_type=jnp.float32)
        obits.append(o / den)

    ob = jnp.concatenate(obits, axis=1)
    vmix = ve0_ref[...] * (1.0 - ob) + ve1_ref[...] * ob
    out_ref[...] = jnp.dot(vmix, wo_ref[...],
                           preferred_element_type=jnp.float32)


def _rosa_single(hs, Wq, Wk, Wv, Wo, ve0, ve1, interpret=False):
    S, HID = hs.shape
    full = lambda shape: pl.BlockSpec(shape, lambda i: (0,) * len(shape))
    return pl.pallas_call(
        _rosa_kernel,
        grid=(S // _BQ,),
        in_specs=[
            full((S, HID)),
            full(Wq.shape), full(Wk.shape), full(Wv.shape), full(Wo.shape),
            full((1, _H * _VB)), full((1, _H * _VB)),
        ],
        out_specs=pl.BlockSpec((_BQ, HID), lambda i: (i, 0)),
        out_shape=jax.ShapeDtypeStruct((S, HID), jnp.float32),
        scratch_shapes=[
            pltpu.VMEM((S, 16), jnp.float32),
            pltpu.VMEM((S, 16), jnp.float32),
            pltpu.VMEM((S, _VB), jnp.float32),
            pltpu.VMEM((S, _VB), jnp.float32),
        ],
        interpret=interpret,
    )(hs, Wq, Wk, Wv, Wo, ve0.reshape(1, -1), ve1.reshape(1, -1))


def kernel(hidden_states, Wq, Wk, Wv, Wo, v_emb0, v_emb1):
    B = hidden_states.shape[0]
    outs = [_rosa_single(hidden_states[b], Wq, Wk, Wv, Wo, v_emb0, v_emb1)
            for b in range(B)]
    return jnp.stack(outs, axis=0)


# speedup vs baseline: 3.9448x; 1.1596x over previous
import jax
import jax.numpy as jnp
from jax.experimental import pallas as pl
from jax.experimental.pallas import tpu as pltpu

_H = 8
_KVH = 2
_QKB = 8
_VB = 16
_TAU = 1.0
_BQ = 256


def _rosa_kernel(hs_ref, wq_ref, wk_ref, wv_ref, wo_ref, ve0_ref, ve1_ref,
                 out_ref, kb0_ref, kb1_ref, vb0_ref, vb1_ref):
    qi = pl.program_id(0)
    S = hs_ref.shape[0]

    @pl.when(qi == 0)
    def _project_kv():
        hs = hs_ref[...]
        kb = jax.nn.sigmoid(
            jnp.dot(hs, wk_ref[...], preferred_element_type=jnp.float32) / _TAU)
        vb = jax.nn.sigmoid(
            jnp.dot(hs, wv_ref[...], preferred_element_type=jnp.float32) / _TAU)
        kpad = jnp.zeros((S, 16 - _QKB - 1), jnp.float32)
        vones = jnp.ones((S, 1), jnp.float32)
        vpad = jnp.zeros((S, 32 - _VB - 1), jnp.float32)
        for g, kref, vref in ((0, kb0_ref, vb0_ref), (1, kb1_ref, vb1_ref)):
            kbg = kb[:, g * _QKB:(g + 1) * _QKB]
            bias = -jnp.sum(kbg, axis=1, keepdims=True) - 16.0
            kref[...] = jnp.concatenate([2.0 * kbg, bias, kpad], axis=1)
            vref[...] = jnp.concatenate(
                [vb[:, g * _VB:(g + 1) * _VB], vones, vpad], axis=1)

    hq = hs_ref[pl.ds(qi * _BQ, _BQ), :]
    qb = jax.nn.sigmoid(
        jnp.dot(hq, wq_ref[...], preferred_element_type=jnp.float32) / _TAU)

    ones = jnp.ones((_BQ, 1), jnp.float32)
    zpad = jnp.zeros((_BQ, 16 - _QKB - 1), jnp.float32)
    qhs, kbs, vbs = [], [], []
    for h in range(_H):
        g = h // (_H // _KVH)
        qhs.append(jnp.concatenate(
            [qb[:, h * _QKB:(h + 1) * _QKB], ones, zpad], axis=1))
        kbs.append(kb0_ref if g == 0 else kb1_ref)
        vbs.append(vb0_ref if g == 0 else vb1_ref)

    dmask = (jax.lax.broadcasted_iota(jnp.int32, (_BQ, _BQ), 1)
             <= jax.lax.broadcasted_iota(jnp.int32, (_BQ, _BQ), 0))
    acc = []
    for h in range(_H):
        kd = kbs[h][pl.ds(qi * _BQ, _BQ), :]
        vd = vbs[h][pl.ds(qi * _BQ, _BQ), :]
        s = jax.lax.dot_general(qhs[h], kd, (((1,), (1,)), ((), ())),
                                preferred_element_type=jnp.float32)
        p = jnp.where(dmask, jnp.exp(s), 0.0)
        acc.append(jnp.dot(p, vd, preferred_element_type=jnp.float32))

    def body(j, carry):
        new = []
        for h in range(_H):
            kd = kbs[h][pl.ds(j * _BQ, _BQ), :]
            vd = vbs[h][pl.ds(j * _BQ, _BQ), :]
            s = jax.lax.dot_general(qhs[h], kd, (((1,), (1,)), ((), ())),
                                    preferred_element_type=jnp.float32)
            p = jnp.exp(s)
            new.append(carry[h] +
                       jnp.dot(p, vd, preferred_element_type=jnp.float32))
        return tuple(new)

    acc = jax.lax.fori_loop(0, qi, body, tuple(acc))

    obits = [a[:, :_VB] / a[:, _VB:_VB + 1] for a in acc]
    ob = jnp.concatenate(obits, axis=1)
    vmix = ve0_ref[...] * (1.0 - ob) + ve1_ref[...] * ob
    out_ref[...] = jnp.dot(vmix, wo_ref[...],
                           preferred_element_type=jnp.float32)


def _rosa_single(hs, Wq, Wk, Wv, Wo, ve0, ve1, interpret=False):
    S, HID = hs.shape
    full = lambda shape: pl.BlockSpec(shape, lambda i: (0,) * len(shape))
    return pl.pallas_call(
        _rosa_kernel,
        grid=(S // _BQ,),
        in_specs=[
            full((S, HID)),
            full(Wq.shape), full(Wk.shape), full(Wv.shape), full(Wo.shape),
            full((1, _H * _VB)), full((1, _H * _VB)),
        ],
        out_specs=pl.BlockSpec((_BQ, HID), lambda i: (i, 0)),
        out_shape=jax.ShapeDtypeStruct((S, HID), jnp.float32),
        scratch_shapes=[
            pltpu.VMEM((S, 16), jnp.float32),
            pltpu.VMEM((S, 16), jnp.float32),
            pltpu.VMEM((S, 32), jnp.float32),
            pltpu.VMEM((S, 32), jnp.float32),
        ],
        interpret=interpret,
    )(hs, Wq, Wk, Wv, Wo, ve0.reshape(1, -1), ve1.reshape(1, -1))


def kernel(hidden_states, Wq, Wk, Wv, Wo, v_emb0, v_emb1):
    B = hidden_states.shape[0]
    outs = [_rosa_single(hidden_states[b], Wq, Wk, Wv, Wo, v_emb0, v_emb1)
            for b in range(B)]
    return jnp.stack(outs, axis=0)
